# baseline (speedup 1.0000x reference)
import jax
import jax.numpy as jnp
from jax.experimental import pallas as pl
from jax.experimental.pallas import tpu as pltpu


def _matmul_body(x_ref, w_ref, s_ref, b_ref, o_ref):
    n_groups, group, bn = w_ref.shape
    w_bf = (w_ref[...].astype(jnp.float32) * s_ref[...]).astype(jnp.bfloat16)
    w_bf = w_bf.reshape(n_groups * group, bn)
    o_ref[...] = jax.lax.dot_general(
        x_ref[...], w_bf,
        dimension_numbers=(((1,), (0,)), ((), ())),
        preferred_element_type=jnp.float32,
    ) + b_ref[...]


def _matmul_body_aliased(x_ref, w_ref, s_ref, b_ref, y_prev_ref, o_ref):
    del y_prev_ref
    _matmul_body(x_ref, w_ref, s_ref, b_ref, o_ref)


def _quant_matmul_half(x_bf, w_t, s_t, b_row, y_prev, *, n_total, ni0, bm, bn):
    m, kdim = x_bf.shape
    n_groups, group, n_half = w_t.shape
    grid = (pl.cdiv(m, bm), pl.cdiv(n_half, bn))
    in_specs = [
        pl.BlockSpec((bm, kdim), lambda mi, ni: (mi, 0)),
        pl.BlockSpec((n_groups, group, bn), lambda mi, ni: (0, 0, ni)),
        pl.BlockSpec((n_groups, 1, bn), lambda mi, ni: (0, 0, ni)),
        pl.BlockSpec((1, bn), lambda mi, ni: (0, ni)),
    ]
    args = [x_bf, w_t, s_t, b_row]
    body = _matmul_body
    aliases = {}
    if y_prev is not None:
        in_specs.append(pl.BlockSpec(memory_space=pl.ANY))
        args.append(y_prev)
        body = _matmul_body_aliased
        aliases = {4: 0}
    return pl.pallas_call(
        body,
        grid=grid,
        in_specs=in_specs,
        out_specs=pl.BlockSpec((bm, bn), lambda mi, ni: (mi, ni + ni0)),
        out_shape=jax.ShapeDtypeStruct((m, n_total), jnp.float32),
        input_output_aliases=aliases,
        compiler_params=pltpu.CompilerParams(
            dimension_semantics=("parallel", "parallel"),
        ),
    )(*args)


def kernel(x, w_q, scales, bias):
    out_f, n_groups, group = w_q.shape
    m, in_f = x.shape
    bm, bn = 2048, 512
    n_cut = 11 * bn
    s2 = scales.reshape(out_f, n_groups)
    x_bf = x.astype(jnp.bfloat16)

    w_t1 = jnp.transpose(w_q[:n_cut], (1, 2, 0))
    s_t1 = s2[:n_cut].T.reshape(n_groups, 1, n_cut)
    b_1 = bias[:n_cut].reshape(1, n_cut)
    w_t2 = jnp.transpose(w_q[n_cut:], (1, 2, 0))
    s_t2 = s2[n_cut:].T.reshape(n_groups, 1, out_f - n_cut)
    b_2 = bias[n_cut:].reshape(1, out_f - n_cut)

    y1 = _quant_matmul_half(x_bf, w_t1, s_t1, b_1, None,
                            n_total=out_f, ni0=0, bm=bm, bn=bn)
    y = _quant_matmul_half(x_bf, w_t2, s_t2, b_2, y1,
                           n_total=out_f, ni0=n_cut // bn, bm=bm, bn=bn)
    return y.astype(x.dtype)


# speedup vs baseline: 1.0809x; 1.0809x over previous
import jax
import jax.numpy as jnp
from jax.experimental import pallas as pl
from jax.experimental.pallas import tpu as pltpu


def _matmul_body(x_ref, w_ref, s_ref, b_ref, o_ref):
    n_groups, group, bn = w_ref.shape
    w_bf = (w_ref[...].astype(jnp.float32) * s_ref[...]).astype(jnp.bfloat16)
    w_bf = w_bf.reshape(n_groups * group, bn)
    o_ref[...] = jax.lax.dot_general(
        x_ref[...], w_bf,
        dimension_numbers=(((1,), (0,)), ((), ())),
        preferred_element_type=jnp.float32,
    ) + b_ref[...]


def _quant_matmul(x_bf, w_t, s_t, b_row, *, bm, bn):
    m, kdim = x_bf.shape
    n_groups, group, n = w_t.shape
    grid = (pl.cdiv(m, bm), pl.cdiv(n, bn))
    return pl.pallas_call(
        _matmul_body,
        grid=grid,
        in_specs=[
            pl.BlockSpec((bm, kdim), lambda mi, ni: (mi, 0)),
            pl.BlockSpec((n_groups, group, bn), lambda mi, ni: (0, 0, ni)),
            pl.BlockSpec((n_groups, 1, bn), lambda mi, ni: (0, 0, ni)),
            pl.BlockSpec((1, bn), lambda mi, ni: (0, ni)),
        ],
        out_specs=pl.BlockSpec((bm, bn), lambda mi, ni: (mi, ni)),
        out_shape=jax.ShapeDtypeStruct((m, n), jnp.float32),
        compiler_params=pltpu.CompilerParams(
            dimension_semantics=("parallel", "parallel"),
        ),
    )(x_bf, w_t, s_t, b_row)


def kernel(x, w_q, scales, bias):
    out_f, n_groups, group = w_q.shape
    m, in_f = x.shape
    w_t = jnp.transpose(w_q, (1, 2, 0))
    s_t = scales.reshape(out_f, n_groups).T.reshape(n_groups, 1, out_f)
    b_row = bias.reshape(1, out_f)
    x_bf = x.astype(jnp.bfloat16)
    y = _quant_matmul(x_bf, w_t, s_t, b_row, bm=2048, bn=512)
    return y.astype(x.dtype)
